# baseline (speedup 1.0000x reference)
import jax
import jax.numpy as jnp

DT = 15
B, H, W = 4, 512, 512


def kernel(x, ylr, msk_lr):
    xlr = x[:, :DT]
    d = (ylr - xlr) * msk_lr.astype(jnp.float32)
    z = jnp.zeros((B, DT, H, W), jnp.float32)
    return d, z, z
